# baseline (speedup 1.0000x reference)
import functools

import jax
import jax.numpy as jnp
from jax import lax
from jax.experimental import pallas as pl
from jax.experimental.pallas import tpu as pltpu
from jax.experimental.pallas import tpu_sc as plsc

NODE_DIM = 64
EDGE_DIM = 32
NC = 2
NS = 16
NW = NC * NS
B = 16384
S = 50
CBW = B // 128 // NW


def _phase(idxT_hbm, tab_hbm, out_hbm, idx_all, rows, trows, gsem, ssem,
           wid, dim):
    nrb = dim // 8
    cb_lo = wid * CBW
    ntask = S * CBW
    iota = lax.broadcasted_iota(jnp.int32, (16,), 0)

    pltpu.sync_copy(idxT_hbm.at[:, pl.ds(cb_lo * 128, CBW * 128)], idx_all)

    def fire(t, b):
        s = t // CBW
        cq = t % CBW
        pltpu.async_copy(tab_hbm.at[idx_all.at[s, pl.ds(cq * 128, 128)]],
                         rows.at[b], gsem.at[b])

    def drain_gather(b):
        pltpu.make_async_copy(tab_hbm.at[pl.ds(0, 128)], rows.at[b],
                              gsem.at[b]).wait()

    def process(t, b):
        drain_gather(b)
        for rb in range(nrb):
            for ri in range(8):
                d = rb * 8 + ri
                col = jnp.full((16,), d, jnp.int32)
                for g in range(8):
                    v = plsc.load_gather(rows.at[b], [iota + g * 16, col])
                    trows[b, rb, ri, pl.ds(g * 16, 16)] = v
        s = t // CBW
        cb = cb_lo + t % CBW
        pltpu.async_copy(trows.at[b], out_hbm.at[s, :, cb], ssem.at[b])

    def drain_store(b):
        pltpu.make_async_copy(trows.at[b], out_hbm.at[0, :, cb_lo],
                              ssem.at[b]).wait()

    fire(0, 0)

    def body(j, carry):
        @pl.when(j >= 1)
        def _():
            drain_store(1)
        fire(2 * j + 1, 1)
        process(2 * j, 0)

        @pl.when(j + 1 < ntask // 2)
        def _():
            drain_store(0)
            fire(2 * j + 2, 0)
        process(2 * j + 1, 1)
        return carry

    lax.fori_loop(0, ntask // 2, body, 0)
    drain_store(0)
    drain_store(1)


def _emb_body(nodesT_hbm, edgesT_hbm, node_tab, edge_tab, out_n, out_e,
              idx_n, idx_e, rows_n, rows_e, trows_n, trows_e, gsem, ssem):
    wid = lax.axis_index("s") * NC + lax.axis_index("c")
    _phase(nodesT_hbm, node_tab, out_n, idx_n, rows_n, trows_n, gsem, ssem,
           wid, NODE_DIM)
    _phase(edgesT_hbm, edge_tab, out_e, idx_e, rows_e, trows_e, gsem, ssem,
           wid, EDGE_DIM)


@jax.jit
def _run(nodes, edges, node_table, edge_table):
    nodes_t = jnp.transpose(nodes)
    edges_t = jnp.transpose(edges)

    mesh = plsc.VectorSubcoreMesh(core_axis_name="c", subcore_axis_name="s")
    run = pl.kernel(
        _emb_body,
        out_type=(
            jax.ShapeDtypeStruct((S, NODE_DIM // 8, B // 128, 8, 128),
                                 jnp.float32),
            jax.ShapeDtypeStruct((S, EDGE_DIM // 8, B // 128, 8, 128),
                                 jnp.float32),
        ),
        mesh=mesh,
        scratch_types=[
            pltpu.VMEM((S, CBW * 128), jnp.int32),
            pltpu.VMEM((S, CBW * 128), jnp.int32),
            pltpu.VMEM((2, 128, NODE_DIM), jnp.float32),
            pltpu.VMEM((2, 128, EDGE_DIM), jnp.float32),
            pltpu.VMEM((2, NODE_DIM // 8, 8, 128), jnp.float32),
            pltpu.VMEM((2, EDGE_DIM // 8, 8, 128), jnp.float32),
            pltpu.SemaphoreType.DMA((2,)),
            pltpu.SemaphoreType.DMA((2,)),
        ],
        compiler_params=pltpu.CompilerParams(use_tc_tiling_on_sc=False,
                                             needs_layout_passes=False),
    )
    o4n, o4e = run(nodes_t, edges_t, node_table, edge_table)

    out_n = jnp.transpose(o4n, (2, 4, 0, 1, 3)).reshape(B, S, NODE_DIM)
    out_e = jnp.transpose(o4e, (2, 4, 0, 1, 3)).reshape(B, S, EDGE_DIM)
    return out_n, out_e


def kernel(nodes, edges, node_table, edge_table):
    return _run(nodes, edges, node_table, edge_table)


# speedup vs baseline: 2.1559x; 2.1559x over previous
import functools

import jax
import jax.numpy as jnp
from jax import lax
from jax.experimental import pallas as pl
from jax.experimental.pallas import tpu as pltpu
from jax.experimental.pallas import tpu_sc as plsc

NODE_DIM = 64
EDGE_DIM = 32
NC = 2
NS = 16
NW = NC * NS
B = 16384
S = 50
CBW = B // 128 // NW


def _phase(idxT_hbm, tab_hbm, out_hbm, idx_all, rows, trows, gsem, ssem,
           wid, dim):
    nrb = dim // 8
    cb_lo = wid * CBW
    ntask = S * CBW
    iota = lax.broadcasted_iota(jnp.int32, (16,), 0)

    pltpu.sync_copy(idxT_hbm.at[:, pl.ds(cb_lo * 128, CBW * 128)], idx_all)

    def fire(t, b):
        s = t // CBW
        cq = t % CBW
        pltpu.async_copy(tab_hbm.at[idx_all.at[s, pl.ds(cq * 128, 128)]],
                         rows.at[b], gsem.at[b])

    def drain_gather(b):
        pltpu.make_async_copy(tab_hbm.at[pl.ds(0, 128)], rows.at[b],
                              gsem.at[b]).wait()

    def process(t, b):
        drain_gather(b)

        def d_body(d, carry):
            dv = d + iota
            dv = jnp.where(dv >= dim, dv - dim, dv)
            rbv = dv // 8
            riv = dv - rbv * 8
            for g in range(8):
                rv = iota + g * 16
                v = plsc.load_gather(rows.at[b], [rv, dv])
                plsc.store_scatter(trows.at[b], [rbv, riv, rv], v)
            return carry

        lax.fori_loop(0, dim, d_body, 0)
        s = t // CBW
        cb = cb_lo + t % CBW
        pltpu.async_copy(trows.at[b], out_hbm.at[s, :, cb], ssem.at[b])

    def drain_store(b):
        pltpu.make_async_copy(trows.at[b], out_hbm.at[0, :, cb_lo],
                              ssem.at[b]).wait()

    fire(0, 0)

    def body(j, carry):
        @pl.when(j >= 1)
        def _():
            drain_store(1)
        fire(2 * j + 1, 1)
        process(2 * j, 0)

        @pl.when(j + 1 < ntask // 2)
        def _():
            drain_store(0)
            fire(2 * j + 2, 0)
        process(2 * j + 1, 1)
        return carry

    lax.fori_loop(0, ntask // 2, body, 0)
    drain_store(0)
    drain_store(1)


def _emb_body(nodesT_hbm, edgesT_hbm, node_tab, edge_tab, out_n, out_e,
              idx_n, idx_e, rows_n, rows_e, trows_n, trows_e, gsem, ssem):
    wid = lax.axis_index("s") * NC + lax.axis_index("c")
    _phase(nodesT_hbm, node_tab, out_n, idx_n, rows_n, trows_n, gsem, ssem,
           wid, NODE_DIM)
    _phase(edgesT_hbm, edge_tab, out_e, idx_e, rows_e, trows_e, gsem, ssem,
           wid, EDGE_DIM)


@jax.jit
def _run(nodes, edges, node_table, edge_table):
    nodes_t = jnp.transpose(nodes)
    edges_t = jnp.transpose(edges)

    mesh = plsc.VectorSubcoreMesh(core_axis_name="c", subcore_axis_name="s")
    run = pl.kernel(
        _emb_body,
        out_type=(
            jax.ShapeDtypeStruct((S, NODE_DIM // 8, B // 128, 8, 128),
                                 jnp.float32),
            jax.ShapeDtypeStruct((S, EDGE_DIM // 8, B // 128, 8, 128),
                                 jnp.float32),
        ),
        mesh=mesh,
        scratch_types=[
            pltpu.VMEM((S, CBW * 128), jnp.int32),
            pltpu.VMEM((S, CBW * 128), jnp.int32),
            pltpu.VMEM((2, 128, NODE_DIM), jnp.float32),
            pltpu.VMEM((2, 128, EDGE_DIM), jnp.float32),
            pltpu.VMEM((2, NODE_DIM // 8, 8, 128), jnp.float32),
            pltpu.VMEM((2, EDGE_DIM // 8, 8, 128), jnp.float32),
            pltpu.SemaphoreType.DMA((2,)),
            pltpu.SemaphoreType.DMA((2,)),
        ],
        compiler_params=pltpu.CompilerParams(use_tc_tiling_on_sc=False,
                                             needs_layout_passes=False),
    )
    o4n, o4e = run(nodes_t, edges_t, node_table, edge_table)

    out_n = jnp.transpose(o4n, (2, 4, 0, 1, 3)).reshape(B, S, NODE_DIM)
    out_e = jnp.transpose(o4e, (2, 4, 0, 1, 3)).reshape(B, S, EDGE_DIM)
    return out_n, out_e


def kernel(nodes, edges, node_table, edge_table):
    return _run(nodes, edges, node_table, edge_table)


# speedup vs baseline: 2.2024x; 1.0216x over previous
import functools

import jax
import jax.numpy as jnp
from jax import lax
from jax.experimental import pallas as pl
from jax.experimental.pallas import tpu as pltpu
from jax.experimental.pallas import tpu_sc as plsc

NODE_DIM = 64
EDGE_DIM = 32
NC = 2
NS = 16
NW = NC * NS
B = 16384
S = 50
CBW = B // 128 // NW


CQ = 2


def _phase(idxT_hbm, tab_hbm, out_hbm, gsem, ssem, wid, dim):
    nrb = dim // 8
    cb_lo = wid * CBW
    ntask = S * CBW // CQ
    iota = lax.broadcasted_iota(jnp.int32, (16,), 0)

    def scoped(idx_all, rows, trows):
        pltpu.sync_copy(idxT_hbm.at[:, pl.ds(cb_lo * 128, CBW * 128)],
                        idx_all)

        def fire(t, b):
            s = t // (CBW // CQ)
            q = t % (CBW // CQ)
            pltpu.async_copy(
                tab_hbm.at[idx_all.at[s, pl.ds(q * CQ * 128, CQ * 128)]],
                rows.at[b], gsem.at[b])

        def drain_gather(b):
            pltpu.make_async_copy(tab_hbm.at[pl.ds(0, CQ * 128)],
                                  rows.at[b], gsem.at[b]).wait()

        def process(t, b):
            drain_gather(b)

            def d_body(d, carry):
                dv = d + iota
                dv = jnp.where(dv >= dim, dv - dim, dv)
                rbv = dv // 8
                riv = dv - rbv * 8
                for cq in range(CQ):
                    cqv = jnp.full((16,), 0, jnp.int32) + cq
                    for g in range(8):
                        rv = iota + g * 16
                        v = plsc.load_gather(rows.at[b],
                                             [rv + cq * 128, dv])
                        plsc.store_scatter(trows.at[b],
                                           [rbv, cqv, riv, rv], v)
                return carry

            lax.fori_loop(0, dim, d_body, 0)
            s = t // (CBW // CQ)
            q = t % (CBW // CQ)
            pltpu.async_copy(
                trows.at[b],
                out_hbm.at[s, :, pl.ds(cb_lo + q * CQ, CQ)], ssem.at[b])

        def drain_store(b):
            pltpu.make_async_copy(trows.at[b],
                                  out_hbm.at[0, :, pl.ds(cb_lo, CQ)],
                                  ssem.at[b]).wait()

        fire(0, 0)

        def body(j, carry):
            @pl.when(j >= 1)
            def _():
                drain_store(1)
            fire(2 * j + 1, 1)
            process(2 * j, 0)

            @pl.when(j + 1 < ntask // 2)
            def _():
                drain_store(0)
                fire(2 * j + 2, 0)
            process(2 * j + 1, 1)
            return carry

        lax.fori_loop(0, ntask // 2, body, 0)
        drain_store(0)
        drain_store(1)

    pl.run_scoped(
        scoped,
        idx_all=pltpu.VMEM((S, CBW * 128), jnp.int32),
        rows=pltpu.VMEM((2, CQ * 128, dim), jnp.float32),
        trows=pltpu.VMEM((2, nrb, CQ, 8, 128), jnp.float32),
    )


def _emb_body(nodesT_hbm, edgesT_hbm, node_tab, edge_tab, out_n, out_e,
              gsem, ssem):
    wid = lax.axis_index("s") * NC + lax.axis_index("c")
    _phase(nodesT_hbm, node_tab, out_n, gsem, ssem, wid, NODE_DIM)
    _phase(edgesT_hbm, edge_tab, out_e, gsem, ssem, wid, EDGE_DIM)


@jax.jit
def _run(nodes, edges, node_table, edge_table):
    nodes_t = jnp.transpose(nodes)
    edges_t = jnp.transpose(edges)

    mesh = plsc.VectorSubcoreMesh(core_axis_name="c", subcore_axis_name="s")
    run = pl.kernel(
        _emb_body,
        out_type=(
            jax.ShapeDtypeStruct((S, NODE_DIM // 8, B // 128, 8, 128),
                                 jnp.float32),
            jax.ShapeDtypeStruct((S, EDGE_DIM // 8, B // 128, 8, 128),
                                 jnp.float32),
        ),
        mesh=mesh,
        scratch_types=[
            pltpu.SemaphoreType.DMA((2,)),
            pltpu.SemaphoreType.DMA((2,)),
        ],
        compiler_params=pltpu.CompilerParams(use_tc_tiling_on_sc=False,
                                             needs_layout_passes=False),
    )
    o4n, o4e = run(nodes_t, edges_t, node_table, edge_table)

    out_n = jnp.transpose(o4n, (2, 4, 0, 1, 3)).reshape(B, S, NODE_DIM)
    out_e = jnp.transpose(o4e, (2, 4, 0, 1, 3)).reshape(B, S, EDGE_DIM)
    return out_n, out_e


def kernel(nodes, edges, node_table, edge_table):
    return _run(nodes, edges, node_table, edge_table)


# speedup vs baseline: 2.9639x; 1.3458x over previous
import functools

import jax
import jax.numpy as jnp
from jax import lax
from jax.experimental import pallas as pl
from jax.experimental.pallas import tpu as pltpu
from jax.experimental.pallas import tpu_sc as plsc

NODE_DIM = 64
EDGE_DIM = 32
NC = 2
NS = 16
NW = NC * NS
B = 16384
S = 50
CBW = B // 128 // NW


CQ = 2


def _phase(idxT_hbm, tab_hbm, out_hbm, gsem, ssem, wid, dim):
    nrb = dim // 8
    cb_lo = wid * CBW
    ntask = S * CBW // CQ
    iota = lax.broadcasted_iota(jnp.int32, (16,), 0)

    def scoped(idx_all, rows, trows):
        pltpu.sync_copy(idxT_hbm.at[:, pl.ds(cb_lo * 128, CBW * 128)],
                        idx_all)

        def fire(t, b):
            s = t // (CBW // CQ)
            q = t % (CBW // CQ)
            pltpu.async_copy(
                tab_hbm.at[idx_all.at[s, pl.ds(q * CQ * 128, CQ * 128)]],
                rows.at[b], gsem.at[b])

        def drain_gather(b):
            pltpu.make_async_copy(tab_hbm.at[pl.ds(0, CQ * 128)],
                                  rows.at[b], gsem.at[b]).wait()

        def process(t, b):
            drain_gather(b)

            def d_body(d, carry):
                dv = d + iota
                dv = jnp.where(dv >= dim, dv - dim, dv)
                rbv = dv // 8
                riv = dv - rbv * 8
                for cq in range(CQ):
                    cqv = jnp.full((16,), 0, jnp.int32) + cq
                    for g in range(8):
                        rv = iota + g * 16
                        v = plsc.load_gather(rows.at[b],
                                             [rv + cq * 128, dv])
                        plsc.store_scatter(trows.at[b],
                                           [rbv, cqv, riv, rv], v)
                return carry

            lax.fori_loop(0, dim, d_body, 0)
            s = t // (CBW // CQ)
            q = t % (CBW // CQ)
            pltpu.async_copy(
                trows.at[b],
                out_hbm.at[s, :, pl.ds(cb_lo + q * CQ, CQ)], ssem.at[b])

        def drain_store(b):
            pltpu.make_async_copy(trows.at[b],
                                  out_hbm.at[0, :, pl.ds(cb_lo, CQ)],
                                  ssem.at[b]).wait()

        fire(0, 0)

        def body(j, carry):
            @pl.when(j >= 1)
            def _():
                drain_store(1)
            fire(2 * j + 1, 1)
            process(2 * j, 0)

            @pl.when(j + 1 < ntask // 2)
            def _():
                drain_store(0)
                fire(2 * j + 2, 0)
            process(2 * j + 1, 1)
            return carry

        lax.fori_loop(0, ntask // 2, body, 0)
        drain_store(0)
        drain_store(1)

    pl.run_scoped(
        scoped,
        idx_all=pltpu.VMEM((S, CBW * 128), jnp.int32),
        rows=pltpu.VMEM((2, CQ * 128, dim), jnp.float32),
        trows=pltpu.VMEM((2, nrb, CQ, 8, 128), jnp.float32),
    )


def _gather_body(idxT_hbm, tab_hbm, out_hbm, gsem, ssem, dim):
    wid = lax.axis_index("s") * NC + lax.axis_index("c")
    _phase(idxT_hbm, tab_hbm, out_hbm, gsem, ssem, wid, dim)


def _sc_call(idx_t, tab, dim):
    mesh = plsc.VectorSubcoreMesh(core_axis_name="c", subcore_axis_name="s")
    run = pl.kernel(
        functools.partial(_gather_body, dim=dim),
        out_type=jax.ShapeDtypeStruct((S, dim // 8, B // 128, 8, 128),
                                      jnp.float32),
        mesh=mesh,
        scratch_types=[
            pltpu.SemaphoreType.DMA((2,)),
            pltpu.SemaphoreType.DMA((2,)),
        ],
        compiler_params=pltpu.CompilerParams(use_tc_tiling_on_sc=False,
                                             needs_layout_passes=False),
    )
    return run(idx_t, tab)


@jax.jit
def _run(nodes, edges, node_table, edge_table):
    nodes_t = jnp.transpose(nodes)
    edges_t = jnp.transpose(edges)

    o4n = _sc_call(nodes_t, node_table, NODE_DIM)
    o4e = _sc_call(edges_t, edge_table, EDGE_DIM)

    out_n = jnp.transpose(o4n, (2, 4, 0, 1, 3)).reshape(B, S, NODE_DIM)
    out_e = jnp.transpose(o4e, (2, 4, 0, 1, 3)).reshape(B, S, EDGE_DIM)
    return out_n, out_e


def kernel(nodes, edges, node_table, edge_table):
    return _run(nodes, edges, node_table, edge_table)
